# baseline (speedup 1.0000x reference)
import functools

import jax
import jax.numpy as jnp
from jax import lax
from jax.experimental import pallas as pl
from jax.experimental.pallas import tpu as pltpu
from jax.experimental.pallas import tpu_sc as plsc

B = 16384
RANK = 32
NW = 32
BPW = B // NW
NCH = BPW // 128
GPC = 128 // 16

TBLK = 256

_mesh = plsc.VectorSubcoreMesh(core_axis_name="c", subcore_axis_name="s")


def _tp_body(x_ref, o_ref):
    for s in range(TBLK // 128):
        for k in range(4):
            j = s * 4 + k
            o_ref[s * 128:(s + 1) * 128, k * 32:(k + 1) * 32] = (
                x_ref[:, j * 128:(j + 1) * 128].T)


def _transpose_table(ut):
    n = ut.shape[1]
    n_super = (n + 511) // 512
    grid = (n_super + 1) // 2
    r = grid * TBLK
    return pl.pallas_call(
        _tp_body,
        grid=(grid,),
        in_specs=[pl.BlockSpec((32, 4 * TBLK), lambda i: (0, i))],
        out_specs=pl.BlockSpec((TBLK, 128), lambda i: (i, 0)),
        out_shape=jax.ShapeDtypeStruct((r, 128), jnp.float32),
    )(ut)


@functools.partial(
    pl.kernel,
    mesh=_mesh,
    out_type=jax.ShapeDtypeStruct((B,), jnp.float32),
    scratch_types=[
        pltpu.VMEM((NCH, 128), jnp.int32),
        pltpu.VMEM((NCH, 128), jnp.int32),
        pltpu.VMEM((NCH, 128), jnp.int32),
        pltpu.VMEM((NCH, 128), jnp.int32),
        pltpu.VMEM((2, 128, 128), jnp.float32),
        pltpu.VMEM((2, 128, 128), jnp.float32),
        pltpu.VMEM((BPW,), jnp.float32),
        pltpu.SemaphoreType.DMA,
        pltpu.SemaphoreType.DMA,
    ],
    compiler_params=pltpu.CompilerParams(needs_layout_passes=False),
)
def _mf_kernel(user_hbm, anime_hbm, u_hbm, v_hbm, out_hbm,
               uidx, aidx, gu, gv, u_rows, v_rows, out_v, sem0, sem1):
    wid = lax.axis_index("s") * 2 + lax.axis_index("c")
    sems = [sem0, sem1]

    pltpu.sync_copy(user_hbm.at[pl.ds(wid * NCH, NCH)], uidx)
    pltpu.sync_copy(anime_hbm.at[pl.ds(wid * NCH, NCH)], aidx)

    for k in range(NCH):
        for g in range(GPC):
            s = pl.ds(g * 16, 16)
            u = uidx[k, s]
            a = aidx[k, s]
            gu[k, s] = lax.bitwise_or(
                lax.shift_left(lax.shift_right_logical(u, 9), 7),
                lax.bitwise_and(u, 127))
            gv[k, s] = lax.bitwise_or(
                lax.shift_left(lax.shift_right_logical(a, 9), 7),
                lax.bitwise_and(a, 127))

    def start_gather(c):
        buf = c % 2
        return (
            pltpu.async_copy(u_hbm.at[gu.at[c]], u_rows.at[buf], sems[buf]),
            pltpu.async_copy(v_hbm.at[gv.at[c]], v_rows.at[buf], sems[buf]),
        )

    lane = lax.iota(jnp.int32, 16)
    three = jnp.full((16,), 3, jnp.int32)

    def compute_chunk(c):
        buf = c % 2

        def group_body(g, carry):
            s = pl.ds(g * 16, 16)
            cbu = lax.shift_left(lax.bitwise_and(
                lax.shift_right_logical(uidx[c, s], 7), three), 5)
            cbv = lax.shift_left(lax.bitwise_and(
                lax.shift_right_logical(aidx[c, s], 7), three), 5)
            row = g * 16 + lane
            acc = jnp.zeros((16,), jnp.float32)
            for j in range(RANK):
                uu = plsc.load_gather(u_rows.at[buf], [row, cbu + j])
                vv = plsc.load_gather(v_rows.at[buf], [row, cbv + j])
                acc = acc + uu * vv
            out_v[pl.ds(c * 128 + g * 16, 16)] = acc
            return carry

        lax.fori_loop(0, GPC, group_body, 0)

    pending = start_gather(0)
    for c in range(NCH):
        nxt = start_gather(c + 1) if c + 1 < NCH else None
        for cp in pending:
            cp.wait()
        compute_chunk(c)
        pending = nxt

    pltpu.sync_copy(out_v, out_hbm.at[pl.ds(wid * BPW, BPW)])


def kernel(user, anime, U, V):
    user = user.astype(jnp.int32).reshape(NW * NCH, 128)
    anime = anime.astype(jnp.int32).reshape(NW * NCH, 128)
    u2 = _transpose_table(U.T)
    v2 = _transpose_table(V.T)
    return _mf_kernel(user, anime, u2, v2)


# speedup vs baseline: 3.9286x; 3.9286x over previous
import functools

import jax
import jax.numpy as jnp
from jax import lax
from jax.experimental import pallas as pl
from jax.experimental.pallas import tpu as pltpu
from jax.experimental.pallas import tpu_sc as plsc

B = 16384
RANK = 32
NW = 32
BPW = B // NW
NCH = BPW // 128
GPC = 128 // 16

TBLK = 2048

_mesh = plsc.VectorSubcoreMesh(core_axis_name="c", subcore_axis_name="s")


def _tp_body(x_ref, o_ref):
    eye = jnp.eye(128, dtype=jnp.bfloat16)
    for s in range(TBLK // 128):
        xs = x_ref[:, s * 512:(s + 1) * 512].astype(jnp.bfloat16)
        w = jnp.concatenate(
            [xs[:, k * 128:(k + 1) * 128] for k in range(4)], axis=0)
        o_ref[s * 128:(s + 1) * 128, :] = jax.lax.dot_general(
            w, eye, (((0,), (0,)), ((), ())),
            preferred_element_type=jnp.float32)


def _transpose_table(ut):
    n = ut.shape[1]
    n_super = (n + 511) // 512
    grid = (n_super + TBLK // 128 - 1) // (TBLK // 128)
    r = grid * TBLK
    return pl.pallas_call(
        _tp_body,
        grid=(grid,),
        in_specs=[pl.BlockSpec((32, 4 * TBLK), lambda i: (0, i))],
        out_specs=pl.BlockSpec((TBLK, 128), lambda i: (i, 0)),
        out_shape=jax.ShapeDtypeStruct((r, 128), jnp.float32),
    )(ut)


@functools.partial(
    pl.kernel,
    mesh=_mesh,
    out_type=jax.ShapeDtypeStruct((B,), jnp.float32),
    scratch_types=[
        pltpu.VMEM((NCH, 128), jnp.int32),
        pltpu.VMEM((NCH, 128), jnp.int32),
        pltpu.VMEM((NCH, 128), jnp.int32),
        pltpu.VMEM((NCH, 128), jnp.int32),
        pltpu.VMEM((2, 128, 128), jnp.float32),
        pltpu.VMEM((2, 128, 128), jnp.float32),
        pltpu.VMEM((BPW,), jnp.float32),
        pltpu.SemaphoreType.DMA,
        pltpu.SemaphoreType.DMA,
    ],
    compiler_params=pltpu.CompilerParams(needs_layout_passes=False),
)
def _mf_kernel(user_hbm, anime_hbm, u_hbm, v_hbm, out_hbm,
               uidx, aidx, gu, gv, u_rows, v_rows, out_v, sem0, sem1):
    wid = lax.axis_index("s") * 2 + lax.axis_index("c")
    sems = [sem0, sem1]

    pltpu.sync_copy(user_hbm.at[pl.ds(wid * NCH, NCH)], uidx)
    pltpu.sync_copy(anime_hbm.at[pl.ds(wid * NCH, NCH)], aidx)

    for k in range(NCH):
        for g in range(GPC):
            s = pl.ds(g * 16, 16)
            u = uidx[k, s]
            a = aidx[k, s]
            gu[k, s] = lax.bitwise_or(
                lax.shift_left(lax.shift_right_logical(u, 9), 7),
                lax.bitwise_and(u, 127))
            gv[k, s] = lax.bitwise_or(
                lax.shift_left(lax.shift_right_logical(a, 9), 7),
                lax.bitwise_and(a, 127))

    def start_gather(c):
        buf = c % 2
        return (
            pltpu.async_copy(u_hbm.at[gu.at[c]], u_rows.at[buf], sems[buf]),
            pltpu.async_copy(v_hbm.at[gv.at[c]], v_rows.at[buf], sems[buf]),
        )

    lane = lax.iota(jnp.int32, 16)
    three = jnp.full((16,), 3, jnp.int32)

    def compute_chunk(c):
        buf = c % 2

        def group_body(g, carry):
            s = pl.ds(g * 16, 16)
            cbu = lax.shift_left(lax.bitwise_and(
                lax.shift_right_logical(uidx[c, s], 7), three), 5)
            cbv = lax.shift_left(lax.bitwise_and(
                lax.shift_right_logical(aidx[c, s], 7), three), 5)
            row = g * 16 + lane
            acc = jnp.zeros((16,), jnp.float32)
            for j in range(RANK):
                uu = plsc.load_gather(u_rows.at[buf], [row, cbu + j])
                vv = plsc.load_gather(v_rows.at[buf], [row, cbv + j])
                acc = acc + uu * vv
            out_v[pl.ds(c * 128 + g * 16, 16)] = acc
            return carry

        lax.fori_loop(0, GPC, group_body, 0)

    pending = start_gather(0)
    for c in range(NCH):
        nxt = start_gather(c + 1) if c + 1 < NCH else None
        for cp in pending:
            cp.wait()
        compute_chunk(c)
        pending = nxt

    pltpu.sync_copy(out_v, out_hbm.at[pl.ds(wid * BPW, BPW)])


def kernel(user, anime, U, V):
    user = user.astype(jnp.int32).reshape(NW * NCH, 128)
    anime = anime.astype(jnp.int32).reshape(NW * NCH, 128)
    u2 = _transpose_table(U.T)
    v2 = _transpose_table(V.T)
    return _mf_kernel(user, anime, u2, v2)


# speedup vs baseline: 5.1154x; 1.3021x over previous
import functools

import jax
import jax.numpy as jnp
from jax import lax
from jax.experimental import pallas as pl
from jax.experimental.pallas import tpu as pltpu
from jax.experimental.pallas import tpu_sc as plsc

B = 16384
RANK = 32
NW = 32
BPW = B // NW
NCH = BPW // 128
GPC = 128 // 16

TBLK = 4096

_mesh = plsc.VectorSubcoreMesh(core_axis_name="c", subcore_axis_name="s")


def _tp_body(x_ref, o_ref):
    eye = jnp.eye(128, dtype=jnp.bfloat16)
    for s2 in range(TBLK // 256):
        xs = x_ref[:, s2 * 1024:(s2 + 1) * 1024].astype(jnp.bfloat16)
        wa = jnp.concatenate(
            [xs[:, k * 128:(k + 1) * 128] for k in range(4)], axis=0)
        wb = jnp.concatenate(
            [xs[:, k * 128:(k + 1) * 128] for k in range(4, 8)], axis=0)
        w = jnp.concatenate([wa, wb], axis=1)
        o_ref[s2 * 256:(s2 + 1) * 256, :] = jax.lax.dot_general(
            w, eye, (((0,), (0,)), ((), ())),
            preferred_element_type=jnp.float32)


def _transpose_table(ut):
    n = ut.shape[1]
    n_super = (n + 511) // 512
    grid = (n_super + TBLK // 128 - 1) // (TBLK // 128)
    r = grid * TBLK
    return pl.pallas_call(
        _tp_body,
        grid=(grid,),
        in_specs=[pl.BlockSpec((32, 4 * TBLK), lambda i: (0, i))],
        out_specs=pl.BlockSpec((TBLK, 128), lambda i: (i, 0)),
        out_shape=jax.ShapeDtypeStruct((r, 128), jnp.float32),
    )(ut)


@functools.partial(
    pl.kernel,
    mesh=_mesh,
    out_type=jax.ShapeDtypeStruct((B,), jnp.float32),
    scratch_types=[
        pltpu.VMEM((NCH, 128), jnp.int32),
        pltpu.VMEM((NCH, 128), jnp.int32),
        pltpu.VMEM((NCH, 128), jnp.int32),
        pltpu.VMEM((NCH, 128), jnp.int32),
        pltpu.VMEM((2, 128, 128), jnp.float32),
        pltpu.VMEM((2, 128, 128), jnp.float32),
        pltpu.VMEM((BPW,), jnp.float32),
        pltpu.SemaphoreType.DMA,
        pltpu.SemaphoreType.DMA,
    ],
    compiler_params=pltpu.CompilerParams(needs_layout_passes=False),
)
def _mf_kernel(user_hbm, anime_hbm, u_hbm, v_hbm, out_hbm,
               uidx, aidx, gu, gv, u_rows, v_rows, out_v, sem0, sem1):
    wid = lax.axis_index("s") * 2 + lax.axis_index("c")
    sems = [sem0, sem1]

    pltpu.sync_copy(user_hbm.at[pl.ds(wid * NCH, NCH)], uidx)
    pltpu.sync_copy(anime_hbm.at[pl.ds(wid * NCH, NCH)], aidx)

    for k in range(NCH):
        for g in range(GPC):
            s = pl.ds(g * 16, 16)
            u = uidx[k, s]
            a = aidx[k, s]
            gu[k, s] = lax.bitwise_or(
                lax.shift_left(lax.shift_right_logical(u, 9), 7),
                lax.bitwise_and(u, 127))
            gv[k, s] = lax.bitwise_or(
                lax.shift_left(lax.shift_right_logical(a, 9), 7),
                lax.bitwise_and(a, 127))

    def start_gather(c):
        buf = c % 2
        return (
            pltpu.async_copy(u_hbm.at[gu.at[c]], u_rows.at[buf], sems[buf]),
            pltpu.async_copy(v_hbm.at[gv.at[c]], v_rows.at[buf], sems[buf]),
        )

    lane = lax.iota(jnp.int32, 16)
    three = jnp.full((16,), 3, jnp.int32)

    def compute_chunk(c):
        buf = c % 2

        def group_body(g, carry):
            s = pl.ds(g * 16, 16)
            cbu = lax.shift_left(lax.bitwise_and(
                lax.shift_right_logical(uidx[c, s], 7), three), 5)
            cbv = lax.shift_left(lax.bitwise_and(
                lax.shift_right_logical(aidx[c, s], 7), three), 5)
            row = g * 16 + lane
            acc = jnp.zeros((16,), jnp.float32)
            for j in range(RANK):
                uu = plsc.load_gather(u_rows.at[buf], [row, cbu + j])
                vv = plsc.load_gather(v_rows.at[buf], [row, cbv + j])
                acc = acc + uu * vv
            out_v[pl.ds(c * 128 + g * 16, 16)] = acc
            return carry

        lax.fori_loop(0, GPC, group_body, 0)

    pending = start_gather(0)
    for c in range(NCH):
        nxt = start_gather(c + 1) if c + 1 < NCH else None
        for cp in pending:
            cp.wait()
        compute_chunk(c)
        pending = nxt

    pltpu.sync_copy(out_v, out_hbm.at[pl.ds(wid * BPW, BPW)])


def kernel(user, anime, U, V):
    user = user.astype(jnp.int32).reshape(NW * NCH, 128)
    anime = anime.astype(jnp.int32).reshape(NW * NCH, 128)
    u2 = _transpose_table(U.T)
    v2 = _transpose_table(V.T)
    return _mf_kernel(user, anime, u2, v2)
